# Initial kernel scaffold; baseline (speedup 1.0000x reference)
import jax
import jax.numpy as jnp
from jax.experimental import pallas as pl


def kernel(rtype_aa, ttype_na, tidx_na, rna, table_aa, table_na, table_type, w_aa_norm, w_na_norm):
    raise NotImplementedError("write your pallas kernel here")


# speedup vs baseline: 1.5089x; 1.5089x over previous
import jax
import jax.numpy as jnp
from jax import lax
from jax.experimental import pallas as pl
from jax.experimental.pallas import tpu as pltpu
from jax.experimental.pallas import tpu_sc as plsc

B, L, C = 64, 2048, 256
N = B * L
NW = 32
ROWS_W = N // NW
CHUNK = 128
NCH = ROWS_W // CHUNK


def _prep_body(raw_aa, raw_na, w_aa, w_na, ttype, rna, norm_aa, norm_na, idx_na):
    def _norm(x, w):
        ms = jnp.mean(x * x, axis=-1, keepdims=True)
        return x * lax.rsqrt(ms + 1e-6) * w

    norm_aa[...] = _norm(raw_aa[...], w_aa[...])
    norm_na[...] = _norm(raw_na[...], w_na[...])
    idx_na[...] = ttype[...] + 16 * rna[...]


_prep = pl.pallas_call(
    _prep_body,
    out_shape=(
        jax.ShapeDtypeStruct((32, C), jnp.float32),
        jax.ShapeDtypeStruct((32, C), jnp.float32),
        jax.ShapeDtypeStruct((B, L), jnp.int32),
    ),
)


def _sc_body(norm_aa, norm_na, idx_aa, idx_na, out_aa, out_na, idx_v, rows_v, gsem):
    wid = lax.axis_index("s") * 2 + lax.axis_index("c")
    base = wid * ROWS_W
    pltpu.sync_copy(idx_aa.at[pl.ds(wid * NCH, NCH)], idx_v.at[0])
    pltpu.sync_copy(idx_na.at[pl.ds(wid * NCH, NCH)], idx_v.at[1])
    for j, (tbl, out) in enumerate(((norm_aa, out_aa), (norm_na, out_na))):
        @pl.loop(0, NCH)
        def _chunk(c):
            pltpu.async_copy(tbl.at[idx_v.at[j, c]], rows_v, gsem).wait()
            pltpu.sync_copy(rows_v, out.at[pl.ds(base + c * CHUNK, CHUNK)])


_sc_gather = pl.kernel(
    _sc_body,
    out_type=(
        jax.ShapeDtypeStruct((N, C), jnp.float32),
        jax.ShapeDtypeStruct((N, C), jnp.float32),
    ),
    mesh=plsc.VectorSubcoreMesh(core_axis_name="c", subcore_axis_name="s"),
    scratch_types=[
        pltpu.VMEM((2, NCH, CHUNK), jnp.int32),
        pltpu.VMEM((CHUNK, C), jnp.float32),
        pltpu.SemaphoreType.DMA,
    ],
)


def kernel(rtype_aa, ttype_na, tidx_na, rna, table_aa, table_na, table_type, w_aa_norm, w_na_norm):
    raw_na = jnp.concatenate(
        [jnp.tile(table_na, (2, 1)), jnp.repeat(table_type, 16, axis=0)], axis=1)
    norm_aa, norm_na, idx_na = _prep(
        table_aa, raw_na, w_aa_norm.reshape(1, C), w_na_norm.reshape(1, C),
        ttype_na, rna.reshape(B, 1))
    out_aa, out_na = _sc_gather(
        norm_aa, norm_na,
        rtype_aa.reshape(N // CHUNK, CHUNK), idx_na.reshape(N // CHUNK, CHUNK))
    return (out_na.reshape(B, L, C), out_aa.reshape(B, L, C))


# speedup vs baseline: 1.5599x; 1.0338x over previous
import jax
import jax.numpy as jnp
from jax import lax
from jax.experimental import pallas as pl
from jax.experimental.pallas import tpu as pltpu
from jax.experimental.pallas import tpu_sc as plsc

B, L, C = 64, 2048, 256
N = B * L
NW = 32
ROWS_W = N // NW
CHUNK = 64
NCHJ = ROWS_W // CHUNK
NBUF = 4


def _prep_body(raw_aa, raw_na, w_aa, w_na, ttype, rna, tbl, idx_na):
    def _norm(x, w):
        ms = jnp.mean(x * x, axis=-1, keepdims=True)
        return x * lax.rsqrt(ms + 1e-6) * w

    tbl[0:32] = _norm(raw_aa[...], w_aa[...])
    tbl[32:64] = _norm(raw_na[...], w_na[...])
    idx_na[...] = ttype[...] + 16 * rna[...] + 32


_prep = pl.pallas_call(
    _prep_body,
    out_shape=(
        jax.ShapeDtypeStruct((64, C), jnp.float32),
        jax.ShapeDtypeStruct((B, L), jnp.int32),
    ),
)


def _sc_body(tbl, idx_aa, idx_na, out_aa, out_na, idx_v, rows_v, *sems):
    gsems, ssems = sems[:NBUF], sems[NBUF:]
    wid = lax.axis_index("s") * 2 + lax.axis_index("c")
    base = wid * ROWS_W
    pltpu.sync_copy(idx_aa.at[pl.ds(wid * NCHJ, NCHJ)], idx_v.at[0])
    pltpu.sync_copy(idx_na.at[pl.ds(wid * NCHJ, NCHJ)], idx_v.at[1])
    for j, out in ((0, out_aa), (1, out_na)):
        for q in range(NBUF - 1):
            pltpu.async_copy(tbl.at[idx_v.at[j, q]], rows_v.at[q], gsems[q])

        @pl.loop(0, NCHJ, step=NBUF)
        def _grp(c):
            for p in range(NBUF):
                cc = c + p
                pn = (p + NBUF - 1) % NBUF
                nxt = cc + NBUF - 1

                @pl.when(nxt < NCHJ)
                def _fire():
                    @pl.when(cc >= 1)
                    def _free():
                        pltpu.make_async_copy(
                            rows_v.at[pn], out.at[pl.ds(base, CHUNK)],
                            ssems[pn]).wait()
                    pltpu.async_copy(tbl.at[idx_v.at[j, nxt]], rows_v.at[pn],
                                     gsems[pn])

                pltpu.make_async_copy(tbl.at[idx_v.at[j, cc]], rows_v.at[p],
                                      gsems[p]).wait()
                pltpu.async_copy(rows_v.at[p],
                                 out.at[pl.ds(base + cc * CHUNK, CHUNK)],
                                 ssems[p])
        for p in range(NBUF):
            pltpu.make_async_copy(rows_v.at[p], out.at[pl.ds(base, CHUNK)],
                                  ssems[p]).wait()


_sc_gather = pl.kernel(
    _sc_body,
    out_type=(
        jax.ShapeDtypeStruct((N, C), jnp.float32),
        jax.ShapeDtypeStruct((N, C), jnp.float32),
    ),
    mesh=plsc.VectorSubcoreMesh(core_axis_name="c", subcore_axis_name="s"),
    scratch_types=[
        pltpu.VMEM((2, NCHJ, CHUNK), jnp.int32),
        pltpu.VMEM((NBUF, CHUNK, C), jnp.float32),
    ] + [pltpu.SemaphoreType.DMA] * (2 * NBUF),
)


def kernel(rtype_aa, ttype_na, tidx_na, rna, table_aa, table_na, table_type, w_aa_norm, w_na_norm):
    raw_na = jnp.concatenate(
        [jnp.tile(table_na, (2, 1)), jnp.repeat(table_type, 16, axis=0)], axis=1)
    tbl, idx_na = _prep(
        table_aa, raw_na, w_aa_norm.reshape(1, C), w_na_norm.reshape(1, C),
        ttype_na, rna.reshape(B, 1))
    out_aa, out_na = _sc_gather(
        tbl, rtype_aa.reshape(N // CHUNK, CHUNK), idx_na.reshape(N // CHUNK, CHUNK))
    return (out_na.reshape(B, L, C), out_aa.reshape(B, L, C))


# speedup vs baseline: 3.8875x; 2.4921x over previous
import jax
import jax.numpy as jnp
from jax import lax
from jax.experimental import pallas as pl
from jax.experimental.pallas import tpu as pltpu
from jax.experimental.pallas import tpu_sc as plsc

B, L, C = 64, 2048, 256
N = B * L
NW = 32
ROWS_W = N // NW
CHUNK = 64
NCHJ = ROWS_W // CHUNK
NBUF = 4
NREP = 16


def _prep_body(raw_aa, raw_na, w_aa, w_na, rtype, ttype, rna, tbl, idx_aa, idx_na):
    def _norm(x, w):
        ms = jnp.mean(x * x, axis=-1, keepdims=True)
        return x * lax.rsqrt(ms + 1e-6) * w

    naa = _norm(raw_aa[...], w_aa[...])
    nna = _norm(raw_na[...], w_na[...])
    for r in range(NREP):
        tbl[64 * r:64 * r + 32] = naa
        tbl[64 * r + 32:64 * r + 64] = nna
    l_ids = lax.broadcasted_iota(jnp.int32, (B, L), 1)
    rep_off = 64 * ((l_ids // CHUNK) % NREP)
    idx_aa[...] = rtype[...] + rep_off
    idx_na[...] = ttype[...] + 16 * rna[...] + 32 + rep_off


_prep = pl.pallas_call(
    _prep_body,
    out_shape=(
        jax.ShapeDtypeStruct((64 * NREP, C), jnp.float32),
        jax.ShapeDtypeStruct((B, L), jnp.int32),
        jax.ShapeDtypeStruct((B, L), jnp.int32),
    ),
)


def _sc_body(tbl, idx_aa, idx_na, out_aa, out_na, idx_v, rows_v, *sems):
    gsems, ssems = sems[:NBUF], sems[NBUF:]
    wid = lax.axis_index("s") * 2 + lax.axis_index("c")
    base = wid * ROWS_W
    pltpu.sync_copy(idx_aa.at[pl.ds(wid * NCHJ, NCHJ)], idx_v.at[0])
    pltpu.sync_copy(idx_na.at[pl.ds(wid * NCHJ, NCHJ)], idx_v.at[1])
    for j, out in ((0, out_aa), (1, out_na)):
        for q in range(NBUF - 1):
            pltpu.async_copy(tbl.at[idx_v.at[j, q]], rows_v.at[q], gsems[q])

        @pl.loop(0, NCHJ, step=NBUF)
        def _grp(c):
            for p in range(NBUF):
                cc = c + p
                pn = (p + NBUF - 1) % NBUF
                nxt = cc + NBUF - 1

                @pl.when(nxt < NCHJ)
                def _fire():
                    @pl.when(cc >= 1)
                    def _free():
                        pltpu.make_async_copy(
                            rows_v.at[pn], out.at[pl.ds(base, CHUNK)],
                            ssems[pn]).wait()
                    pltpu.async_copy(tbl.at[idx_v.at[j, nxt]], rows_v.at[pn],
                                     gsems[pn])

                pltpu.make_async_copy(tbl.at[idx_v.at[j, cc]], rows_v.at[p],
                                      gsems[p]).wait()
                pltpu.async_copy(rows_v.at[p],
                                 out.at[pl.ds(base + cc * CHUNK, CHUNK)],
                                 ssems[p])
        for p in range(NBUF):
            pltpu.make_async_copy(rows_v.at[p], out.at[pl.ds(base, CHUNK)],
                                  ssems[p]).wait()


_sc_gather = pl.kernel(
    _sc_body,
    out_type=(
        jax.ShapeDtypeStruct((N, C), jnp.float32),
        jax.ShapeDtypeStruct((N, C), jnp.float32),
    ),
    mesh=plsc.VectorSubcoreMesh(core_axis_name="c", subcore_axis_name="s"),
    scratch_types=[
        pltpu.VMEM((2, NCHJ, CHUNK), jnp.int32),
        pltpu.VMEM((NBUF, CHUNK, C), jnp.float32),
    ] + [pltpu.SemaphoreType.DMA] * (2 * NBUF),
)


def kernel(rtype_aa, ttype_na, tidx_na, rna, table_aa, table_na, table_type, w_aa_norm, w_na_norm):
    raw_na = jnp.concatenate(
        [jnp.tile(table_na, (2, 1)), jnp.repeat(table_type, 16, axis=0)], axis=1)
    tbl, idx_aa, idx_na = _prep(
        table_aa, raw_na, w_aa_norm.reshape(1, C), w_na_norm.reshape(1, C),
        rtype_aa, ttype_na, rna.reshape(B, 1))
    out_aa, out_na = _sc_gather(
        tbl, idx_aa.reshape(N // CHUNK, CHUNK), idx_na.reshape(N // CHUNK, CHUNK))
    return (out_na.reshape(B, L, C), out_aa.reshape(B, L, C))


# speedup vs baseline: 4.9427x; 1.2715x over previous
import jax
import jax.numpy as jnp
from jax import lax
from jax.experimental import pallas as pl
from jax.experimental.pallas import tpu as pltpu
from jax.experimental.pallas import tpu_sc as plsc

B, L, C = 64, 2048, 256
N = B * L
NW = 32
ROWS_W = N // NW
CHUNK = 64
NCHJ = ROWS_W // CHUNK
NBUF = 4
NREP = 32


def _prep_body(raw_aa, raw_na, w_aa, w_na, rtype, ttype, rna, tbl, idx_aa, idx_na):
    def _norm(x, w):
        ms = jnp.mean(x * x, axis=-1, keepdims=True)
        return x * lax.rsqrt(ms + 1e-6) * w

    naa = _norm(raw_aa[...], w_aa[...])
    nna = _norm(raw_na[...], w_na[...])
    for r in range(NREP):
        tbl[64 * r:64 * r + 32] = naa
        tbl[64 * r + 32:64 * r + 64] = nna
    l_ids = lax.broadcasted_iota(jnp.int32, (B, L), 1)
    b_ids = lax.broadcasted_iota(jnp.int32, (B, L), 0)
    rep_off = 64 * ((b_ids + l_ids // CHUNK) % NREP)
    idx_aa[...] = rtype[...] + rep_off
    idx_na[...] = ttype[...] + 16 * rna[...] + 32 + rep_off


_prep = pl.pallas_call(
    _prep_body,
    out_shape=(
        jax.ShapeDtypeStruct((64 * NREP, C), jnp.float32),
        jax.ShapeDtypeStruct((B, L), jnp.int32),
        jax.ShapeDtypeStruct((B, L), jnp.int32),
    ),
)


def _sc_body(tbl, idx_aa, idx_na, out_aa, out_na, idx_v, rows_v, *sems):
    gsems, ssems = sems[:NBUF], sems[NBUF:]
    wid = lax.axis_index("s") * 2 + lax.axis_index("c")
    base = wid * ROWS_W
    pltpu.sync_copy(idx_aa.at[pl.ds(wid * NCHJ, NCHJ)], idx_v.at[0])
    pltpu.sync_copy(idx_na.at[pl.ds(wid * NCHJ, NCHJ)], idx_v.at[1])
    for j, out in ((0, out_aa), (1, out_na)):
        for q in range(NBUF - 1):
            pltpu.async_copy(tbl.at[idx_v.at[j, q]], rows_v.at[q], gsems[q])

        @pl.loop(0, NCHJ, step=NBUF)
        def _grp(c):
            for p in range(NBUF):
                cc = c + p
                pn = (p + NBUF - 1) % NBUF
                nxt = cc + NBUF - 1

                @pl.when(nxt < NCHJ)
                def _fire():
                    @pl.when(cc >= 1)
                    def _free():
                        pltpu.make_async_copy(
                            rows_v.at[pn], out.at[pl.ds(base, CHUNK)],
                            ssems[pn]).wait()
                    pltpu.async_copy(tbl.at[idx_v.at[j, nxt]], rows_v.at[pn],
                                     gsems[pn])

                pltpu.make_async_copy(tbl.at[idx_v.at[j, cc]], rows_v.at[p],
                                      gsems[p]).wait()
                pltpu.async_copy(rows_v.at[p],
                                 out.at[pl.ds(base + cc * CHUNK, CHUNK)],
                                 ssems[p])
        for p in range(NBUF):
            pltpu.make_async_copy(rows_v.at[p], out.at[pl.ds(base, CHUNK)],
                                  ssems[p]).wait()


_sc_gather = pl.kernel(
    _sc_body,
    out_type=(
        jax.ShapeDtypeStruct((N, C), jnp.float32),
        jax.ShapeDtypeStruct((N, C), jnp.float32),
    ),
    mesh=plsc.VectorSubcoreMesh(core_axis_name="c", subcore_axis_name="s"),
    scratch_types=[
        pltpu.VMEM((2, NCHJ, CHUNK), jnp.int32),
        pltpu.VMEM((NBUF, CHUNK, C), jnp.float32),
    ] + [pltpu.SemaphoreType.DMA] * (2 * NBUF),
)


def kernel(rtype_aa, ttype_na, tidx_na, rna, table_aa, table_na, table_type, w_aa_norm, w_na_norm):
    raw_na = jnp.concatenate(
        [jnp.tile(table_na, (2, 1)), jnp.repeat(table_type, 16, axis=0)], axis=1)
    tbl, idx_aa, idx_na = _prep(
        table_aa, raw_na, w_aa_norm.reshape(1, C), w_na_norm.reshape(1, C),
        rtype_aa, ttype_na, rna.reshape(B, 1))
    out_aa, out_na = _sc_gather(
        tbl, idx_aa.reshape(N // CHUNK, CHUNK), idx_na.reshape(N // CHUNK, CHUNK))
    return (out_na.reshape(B, L, C), out_aa.reshape(B, L, C))


# speedup vs baseline: 5.5997x; 1.1329x over previous
import jax
import jax.numpy as jnp
from jax import lax
from jax.experimental import pallas as pl
from jax.experimental.pallas import tpu as pltpu
from jax.experimental.pallas import tpu_sc as plsc

B, L, C = 64, 2048, 256
N = B * L
NW = 32
ROWS_W = N // NW
CHUNK = 64
NCHJ = ROWS_W // CHUNK
NBUF = 4
NREP = 64


def _prep_body(raw_aa, raw_na, w_aa, w_na, rtype, ttype, rna, tbl, idx_aa, idx_na):
    def _norm(x, w):
        ms = jnp.mean(x * x, axis=-1, keepdims=True)
        return x * lax.rsqrt(ms + 1e-6) * w

    naa = _norm(raw_aa[...], w_aa[...])
    nna = _norm(raw_na[...], w_na[...])
    for r in range(NREP):
        tbl[64 * r:64 * r + 32] = naa
        tbl[64 * r + 32:64 * r + 64] = nna
    l_ids = lax.broadcasted_iota(jnp.int32, (B, L), 1)
    b_ids = lax.broadcasted_iota(jnp.int32, (B, L), 0)
    rep_off = 64 * ((b_ids + l_ids) % NREP)
    idx_aa[...] = rtype[...] + rep_off
    idx_na[...] = ttype[...] + 16 * rna[...] + 32 + rep_off


_prep = pl.pallas_call(
    _prep_body,
    out_shape=(
        jax.ShapeDtypeStruct((64 * NREP, C), jnp.float32),
        jax.ShapeDtypeStruct((B, L), jnp.int32),
        jax.ShapeDtypeStruct((B, L), jnp.int32),
    ),
)


def _sc_body(tbl, idx_aa, idx_na, out_aa, out_na, idx_v, rows_v, *sems):
    gsems, ssems = sems[:NBUF], sems[NBUF:]
    wid = lax.axis_index("s") * 2 + lax.axis_index("c")
    base = wid * ROWS_W
    pltpu.sync_copy(idx_aa.at[pl.ds(wid * NCHJ, NCHJ)], idx_v.at[0])
    pltpu.sync_copy(idx_na.at[pl.ds(wid * NCHJ, NCHJ)], idx_v.at[1])
    for j, out in ((0, out_aa), (1, out_na)):
        for q in range(NBUF - 1):
            pltpu.async_copy(tbl.at[idx_v.at[j, q]], rows_v.at[q], gsems[q])

        @pl.loop(0, NCHJ, step=NBUF)
        def _grp(c):
            for p in range(NBUF):
                cc = c + p
                pn = (p + NBUF - 1) % NBUF
                nxt = cc + NBUF - 1

                @pl.when(nxt < NCHJ)
                def _fire():
                    @pl.when(cc >= 1)
                    def _free():
                        pltpu.make_async_copy(
                            rows_v.at[pn], out.at[pl.ds(base, CHUNK)],
                            ssems[pn]).wait()
                    pltpu.async_copy(tbl.at[idx_v.at[j, nxt]], rows_v.at[pn],
                                     gsems[pn])

                pltpu.make_async_copy(tbl.at[idx_v.at[j, cc]], rows_v.at[p],
                                      gsems[p]).wait()
                pltpu.async_copy(rows_v.at[p],
                                 out.at[pl.ds(base + cc * CHUNK, CHUNK)],
                                 ssems[p])
        for p in range(NBUF):
            pltpu.make_async_copy(rows_v.at[p], out.at[pl.ds(base, CHUNK)],
                                  ssems[p]).wait()


_sc_gather = pl.kernel(
    _sc_body,
    out_type=(
        jax.ShapeDtypeStruct((N, C), jnp.float32),
        jax.ShapeDtypeStruct((N, C), jnp.float32),
    ),
    mesh=plsc.VectorSubcoreMesh(core_axis_name="c", subcore_axis_name="s"),
    scratch_types=[
        pltpu.VMEM((2, NCHJ, CHUNK), jnp.int32),
        pltpu.VMEM((NBUF, CHUNK, C), jnp.float32),
    ] + [pltpu.SemaphoreType.DMA] * (2 * NBUF),
)


def kernel(rtype_aa, ttype_na, tidx_na, rna, table_aa, table_na, table_type, w_aa_norm, w_na_norm):
    raw_na = jnp.concatenate(
        [jnp.tile(table_na, (2, 1)), jnp.repeat(table_type, 16, axis=0)], axis=1)
    tbl, idx_aa, idx_na = _prep(
        table_aa, raw_na, w_aa_norm.reshape(1, C), w_na_norm.reshape(1, C),
        rtype_aa, ttype_na, rna.reshape(B, 1))
    out_aa, out_na = _sc_gather(
        tbl, idx_aa.reshape(N // CHUNK, CHUNK), idx_na.reshape(N // CHUNK, CHUNK))
    return (out_na.reshape(B, L, C), out_aa.reshape(B, L, C))


# speedup vs baseline: 6.0242x; 1.0758x over previous
import jax
import jax.numpy as jnp
from jax import lax
from jax.experimental import pallas as pl
from jax.experimental.pallas import tpu as pltpu
from jax.experimental.pallas import tpu_sc as plsc

B, L, C = 64, 2048, 256
N = B * L
NW = 32
ROWS_W = N // NW
CHUNK = 64
NCHJ = ROWS_W // CHUNK
NBUF = 4
NREP = 128


def _prep_body(raw_aa, raw_na, w_aa, w_na, rtype, ttype, rna, tbl, idx_aa, idx_na):
    def _norm(x, w):
        ms = jnp.mean(x * x, axis=-1, keepdims=True)
        return x * lax.rsqrt(ms + 1e-6) * w

    naa = _norm(raw_aa[...], w_aa[...])
    nna = _norm(raw_na[...], w_na[...])
    for r in range(NREP):
        tbl[64 * r:64 * r + 32] = naa
        tbl[64 * r + 32:64 * r + 64] = nna
    l_ids = lax.broadcasted_iota(jnp.int32, (B, L), 1)
    b_ids = lax.broadcasted_iota(jnp.int32, (B, L), 0)
    rep_off = 64 * ((b_ids + l_ids) % NREP)
    idx_aa[...] = rtype[...] + rep_off
    idx_na[...] = ttype[...] + 16 * rna[...] + 32 + rep_off


_prep = pl.pallas_call(
    _prep_body,
    out_shape=(
        jax.ShapeDtypeStruct((64 * NREP, C), jnp.float32),
        jax.ShapeDtypeStruct((B, L), jnp.int32),
        jax.ShapeDtypeStruct((B, L), jnp.int32),
    ),
)


def _sc_body(tbl, idx_aa, idx_na, out_aa, out_na, idx_v, rows_v, *sems):
    gsems, ssems = sems[:NBUF], sems[NBUF:]
    wid = lax.axis_index("s") * 2 + lax.axis_index("c")
    base = wid * ROWS_W
    pltpu.sync_copy(idx_aa.at[pl.ds(wid * NCHJ, NCHJ)], idx_v.at[0])
    pltpu.sync_copy(idx_na.at[pl.ds(wid * NCHJ, NCHJ)], idx_v.at[1])
    for j, out in ((0, out_aa), (1, out_na)):
        for q in range(NBUF - 1):
            pltpu.async_copy(tbl.at[idx_v.at[j, q]], rows_v.at[q], gsems[q])

        @pl.loop(0, NCHJ, step=NBUF)
        def _grp(c):
            for p in range(NBUF):
                cc = c + p
                pn = (p + NBUF - 1) % NBUF
                nxt = cc + NBUF - 1

                @pl.when(nxt < NCHJ)
                def _fire():
                    @pl.when(cc >= 1)
                    def _free():
                        pltpu.make_async_copy(
                            rows_v.at[pn], out.at[pl.ds(base, CHUNK)],
                            ssems[pn]).wait()
                    pltpu.async_copy(tbl.at[idx_v.at[j, nxt]], rows_v.at[pn],
                                     gsems[pn])

                pltpu.make_async_copy(tbl.at[idx_v.at[j, cc]], rows_v.at[p],
                                      gsems[p]).wait()
                pltpu.async_copy(rows_v.at[p],
                                 out.at[pl.ds(base + cc * CHUNK, CHUNK)],
                                 ssems[p])
        for p in range(NBUF):
            pltpu.make_async_copy(rows_v.at[p], out.at[pl.ds(base, CHUNK)],
                                  ssems[p]).wait()


_sc_gather = pl.kernel(
    _sc_body,
    out_type=(
        jax.ShapeDtypeStruct((N, C), jnp.float32),
        jax.ShapeDtypeStruct((N, C), jnp.float32),
    ),
    mesh=plsc.VectorSubcoreMesh(core_axis_name="c", subcore_axis_name="s"),
    scratch_types=[
        pltpu.VMEM((2, NCHJ, CHUNK), jnp.int32),
        pltpu.VMEM((NBUF, CHUNK, C), jnp.float32),
    ] + [pltpu.SemaphoreType.DMA] * (2 * NBUF),
)


def kernel(rtype_aa, ttype_na, tidx_na, rna, table_aa, table_na, table_type, w_aa_norm, w_na_norm):
    raw_na = jnp.concatenate(
        [jnp.tile(table_na, (2, 1)), jnp.repeat(table_type, 16, axis=0)], axis=1)
    tbl, idx_aa, idx_na = _prep(
        table_aa, raw_na, w_aa_norm.reshape(1, C), w_na_norm.reshape(1, C),
        rtype_aa, ttype_na, rna.reshape(B, 1))
    out_aa, out_na = _sc_gather(
        tbl, idx_aa.reshape(N // CHUNK, CHUNK), idx_na.reshape(N // CHUNK, CHUNK))
    return (out_na.reshape(B, L, C), out_aa.reshape(B, L, C))
